# Initial kernel scaffold; baseline (speedup 1.0000x reference)
import functools

import jax
import jax.numpy as jnp
from jax.experimental import pallas as pl
from jax.experimental.pallas import tpu as pltpu


def _layernorm(x, g, b):
    m = jnp.mean(x, axis=-1, keepdims=True)
    d = x - m
    v = jnp.mean(d * d, axis=-1, keepdims=True)
    return d * jax.lax.rsqrt(v + 1e-5) * g + b


def _lrelu(x):
    return jnp.where(x >= 0, x, 0.2 * x)


def _elu(x):
    return jnp.where(x > 0, x, jnp.expm1(x))


def _dot(a, b):
    return jnp.dot(a, b, preferred_element_type=jnp.float32)


def _gat_step(x_ref, ps_ref, pd_ref, pdt_ref,
              wl0, wr0, ae0, b0, g0, bl0,
              wl1, wr1, ae1, b1, g1, bl1,
              wl2, wr2, ae2, b2, g2, bl2,
              hexp_ref, wf_ref, bf_ref, wb_ref, bb_ref,
              gxf_ref, gxb_ref):
    x = x_ref[0, 0]
    ps = ps_ref[...]
    pd = pd_ref[...]
    pdt = pdt_ref[...]
    hexp = hexp_ref[...]
    layers = ((wl0, wr0, ae0, b0, g0, bl0),
              (wl1, wr1, ae1, b1, g1, bl1),
              (wl2, wr2, ae2, b2, g2, bl2))
    for (wl, wr, ae, bias, lg, lb) in layers:
        xl = _dot(x, wl[...])
        xr = _dot(x, wr[...])
        zl = _dot(ps, xl)
        zr = _dot(pd, xr)
        z = _lrelu(zl + zr)
        e = _dot(z, ae[...])
        s = jnp.max(e)
        ex = jnp.exp(e - s)
        den = _dot(pdt, ex)
        dend = _dot(pd, den)
        alpha = ex / (dend + 1e-16)
        aexp = _dot(alpha, hexp)
        out = _dot(pdt, zl * aexp) + bias[...]
        x = _elu(_layernorm(out, lg[...], lb[...]))
    gxf_ref[0, 0] = _dot(x, wf_ref[...]) + bf_ref[...]
    gxb_ref[0, 0] = _dot(x, wb_ref[...]) + bb_ref[...]


def _lstm_dirs(gxf_ref, gxb_ref, whf_ref, whb_ref, hf_s, cf_s, hb_s, cb_s,
               lh):
    i = pl.program_id(0)

    @pl.when(i == 0)
    def _init():
        z = jnp.zeros_like(hf_s)
        hf_s[...] = z
        cf_s[...] = z
        hb_s[...] = z
        cb_s[...] = z

    def one_dir(gx, h, c, whh):
        g = gx + _dot(h, whh)
        ig = jax.nn.sigmoid(g[:, 0 * lh:1 * lh])
        fg = jax.nn.sigmoid(g[:, 1 * lh:2 * lh])
        gg = jnp.tanh(g[:, 2 * lh:3 * lh])
        og = jax.nn.sigmoid(g[:, 3 * lh:4 * lh])
        c2 = fg * c + ig * gg
        h2 = og * jnp.tanh(c2)
        return h2, c2

    h2f, c2f = one_dir(gxf_ref[0], hf_s[...], cf_s[...], whf_ref[...])
    hf_s[...] = h2f
    cf_s[...] = c2f
    h2b, c2b = one_dir(gxb_ref[0], hb_s[...], cb_s[...], whb_ref[...])
    hb_s[...] = h2b
    cb_s[...] = c2b
    return h2f, h2b


def _lstm_step(gxf_ref, gxb_ref, whf_ref, whb_ref, hf_out, hb_out,
               hf_s, cf_s, hb_s, cb_s, *, lh):
    h2f, h2b = _lstm_dirs(gxf_ref, gxb_ref, whf_ref, whb_ref,
                          hf_s, cf_s, hb_s, cb_s, lh)
    hf_out[0] = h2f
    hb_out[0] = h2b


def _lstm_last_step(gxf_ref, gxb_ref, whf_ref, whb_ref, hf_out, hb_out,
                    hf_s, cf_s, hb_s, cb_s, *, lh):
    h2f, h2b = _lstm_dirs(gxf_ref, gxb_ref, whf_ref, whb_ref,
                          hf_s, cf_s, hb_s, cb_s, lh)
    hf_out[...] = h2f

    @pl.when(pl.program_id(0) == 0)
    def _store_bwd():
        hb_out[...] = h2b


def _mm_step(x_ref, w_ref, b_ref, o_ref):
    o_ref[0] = _dot(x_ref[0], w_ref[...]) + b_ref[...]


def _heads_step(t_ref, w1, b1, lg, lb, w2, b2, dw1, db1, dw2, db2,
                h_out, d_out):
    t = t_ref[...]
    h = jax.nn.relu(_dot(t, w1[...]) + b1[...])
    h = _layernorm(h, lg[...], lb[...])
    h_out[...] = jax.nn.sigmoid(_dot(h, w2[...]) + b2[...])
    d = jax.nn.relu(_dot(t, dw1[...]) + db1[...])
    d_out[...] = jnp.tanh(_dot(d, dw2[...]) + db2[...])


def _row(v):
    return v.reshape(1, -1)


@jax.jit
def kernel(x_sequence, edge_index, params):
    B, T, N, F = x_sequence.shape
    HEADS, HC = params['gat0_att'].shape
    H = HEADS * HC
    LH = params['lstm0_fwd_Whh'].shape[1]
    G = 4 * LH
    BN = B * N
    f32 = jnp.float32

    ar = jnp.arange(N, dtype=edge_index.dtype)
    src = jnp.concatenate([edge_index[0], ar])
    dst = jnp.concatenate([edge_index[1], ar])
    Ne = src.shape[0]
    cols = ar[None, :]
    Ps = (src[:, None] == cols).astype(f32)
    Pd = (dst[:, None] == cols).astype(f32)
    PdT = Pd.T

    eye = jnp.eye(HEADS, dtype=f32)
    hexp = jnp.repeat(eye, HC, axis=1)
    att_embs = [
        (params['gat%d_att' % i][:, :, None] * eye[:, None, :]).reshape(H, HEADS)
        for i in range(3)
    ]

    def full(shape):
        return pl.BlockSpec(shape, lambda t, b: tuple(0 for _ in shape))

    gat_in_specs = [
        pl.BlockSpec((1, 1, N, F), lambda t, b: (b, t, 0, 0)),
        full((Ne, N)), full((Ne, N)), full((N, Ne)),
    ]
    gat_args = [x_sequence, Ps, Pd, PdT]
    dins = [F, H, H]
    for i in range(3):
        gat_args += [params['gat%d_Wl' % i], params['gat%d_Wr' % i],
                     att_embs[i], _row(params['gat%d_b' % i]),
                     _row(params['ln%d_g' % i]), _row(params['ln%d_b' % i])]
        gat_in_specs += [full((dins[i], H)), full((dins[i], H)),
                         full((H, HEADS)), full((1, H)), full((1, H)),
                         full((1, H))]
    gat_args += [
        hexp,
        params['lstm0_fwd_Wih'].T,
        _row(params['lstm0_fwd_bih'] + params['lstm0_fwd_bhh']),
        params['lstm0_bwd_Wih'].T,
        _row(params['lstm0_bwd_bih'] + params['lstm0_bwd_bhh']),
    ]
    gat_in_specs += [full((HEADS, H)), full((H, G)), full((1, G)),
                     full((H, G)), full((1, G))]

    gxf, gxb = pl.pallas_call(
        _gat_step,
        grid=(T, B),
        in_specs=gat_in_specs,
        out_specs=[pl.BlockSpec((1, 1, N, G), lambda t, b: (t, b, 0, 0))] * 2,
        out_shape=[jax.ShapeDtypeStruct((T, B, N, G), f32)] * 2,
        compiler_params=pltpu.CompilerParams(
            dimension_semantics=("parallel", "parallel")),
    )(*gat_args)

    gxf = gxf.reshape(T, BN, G)
    gxb = gxb.reshape(T, BN, G)

    lstm0 = functools.partial(_lstm_step, lh=LH)
    hf, hb = pl.pallas_call(
        lstm0,
        grid=(T,),
        in_specs=[
            pl.BlockSpec((1, BN, G), lambda t: (t, 0, 0)),
            pl.BlockSpec((1, BN, G), lambda t: (T - 1 - t, 0, 0)),
            pl.BlockSpec((LH, G), lambda t: (0, 0)),
            pl.BlockSpec((LH, G), lambda t: (0, 0)),
        ],
        out_specs=[
            pl.BlockSpec((1, BN, LH), lambda t: (t, 0, 0)),
            pl.BlockSpec((1, BN, LH), lambda t: (T - 1 - t, 0, 0)),
        ],
        out_shape=[jax.ShapeDtypeStruct((T, BN, LH), f32)] * 2,
        scratch_shapes=[pltpu.VMEM((BN, LH), f32)] * 4,
    )(gxf, gxb, params['lstm0_fwd_Whh'].T, params['lstm0_bwd_Whh'].T)

    x1 = jnp.concatenate([hf, hb], axis=-1)
    w1cat = jnp.concatenate(
        [params['lstm1_fwd_Wih'].T, params['lstm1_bwd_Wih'].T], axis=-1)
    b1cat = jnp.concatenate([
        params['lstm1_fwd_bih'] + params['lstm1_fwd_bhh'],
        params['lstm1_bwd_bih'] + params['lstm1_bwd_bhh']]).reshape(1, 2 * G)
    gx1 = pl.pallas_call(
        _mm_step,
        grid=(T,),
        in_specs=[
            pl.BlockSpec((1, BN, 2 * LH), lambda t: (t, 0, 0)),
            pl.BlockSpec((2 * LH, 2 * G), lambda t: (0, 0)),
            pl.BlockSpec((1, 2 * G), lambda t: (0, 0)),
        ],
        out_specs=pl.BlockSpec((1, BN, 2 * G), lambda t: (t, 0, 0)),
        out_shape=jax.ShapeDtypeStruct((T, BN, 2 * G), f32),
        compiler_params=pltpu.CompilerParams(
            dimension_semantics=("parallel",)),
    )(x1, w1cat, b1cat)

    lstm1 = functools.partial(_lstm_last_step, lh=LH)
    hf1, hb1 = pl.pallas_call(
        lstm1,
        grid=(T,),
        in_specs=[
            pl.BlockSpec((1, BN, G), lambda t: (t, 0, 0)),
            pl.BlockSpec((1, BN, G), lambda t: (T - 1 - t, 0, 1)),
            pl.BlockSpec((LH, G), lambda t: (0, 0)),
            pl.BlockSpec((LH, G), lambda t: (0, 0)),
        ],
        out_specs=[
            pl.BlockSpec((BN, LH), lambda t: (0, 0)),
            pl.BlockSpec((BN, LH), lambda t: (0, 0)),
        ],
        out_shape=[jax.ShapeDtypeStruct((BN, LH), f32)] * 2,
        scratch_shapes=[pltpu.VMEM((BN, LH), f32)] * 4,
    )(gx1, gx1, params['lstm1_fwd_Whh'].T, params['lstm1_bwd_Whh'].T)

    temporal = jnp.concatenate([hf1, hb1], axis=-1)
    health, deg = pl.pallas_call(
        _heads_step,
        out_shape=[jax.ShapeDtypeStruct((BN, 1), f32)] * 2,
    )(temporal,
      params['hh_W1'], _row(params['hh_b1']),
      _row(params['hh_lng']), _row(params['hh_lnb']),
      params['hh_W2'], _row(params['hh_b2']),
      params['dh_W1'], _row(params['dh_b1']),
      params['dh_W2'], _row(params['dh_b2']))

    return health.reshape(B, N), deg.reshape(B, N)


# speedup vs baseline: 75.5454x; 75.5454x over previous
import functools

import jax
import jax.numpy as jnp
from jax.experimental import pallas as pl
from jax.experimental.pallas import tpu as pltpu


def _layernorm(x, g, b):
    m = jnp.mean(x, axis=-1, keepdims=True)
    d = x - m
    v = jnp.mean(d * d, axis=-1, keepdims=True)
    return d * jax.lax.rsqrt(v + 1e-5) * g + b


def _lrelu(x):
    return jnp.where(x >= 0, x, 0.2 * x)


def _elu(x):
    return jnp.where(x > 0, x, jnp.exp(x) - 1.0)


def _dot(a, b):
    return jnp.dot(a, b, preferred_element_type=jnp.float32)


def _gat_step(x_ref, ps_ref, pd_ref, pdt_ref,
              wl0, wr0, ae0, b0, g0, bl0,
              wl1, wr1, ae1, b1, g1, bl1,
              wl2, wr2, ae2, b2, g2, bl2,
              hexp_ref, wf_ref, bf_ref, wb_ref, bb_ref,
              gxf_ref, gxb_ref, *, tc, kq):
    x = x_ref[0, 0]
    ps = ps_ref[...]
    pd = pd_ref[...]
    pdt = pdt_ref[...]
    hexp = hexp_ref[...]
    layers = ((wl0, wr0, ae0, b0, g0, bl0),
              (wl1, wr1, ae1, b1, g1, bl1),
              (wl2, wr2, ae2, b2, g2, bl2))
    for (wl, wr, ae, bias, lg, lb) in layers:
        xl = _dot(x, wl[...])
        xr = _dot(x, wr[...])
        zl = _dot(ps, xl)
        zr = _dot(pd, xr)
        z = _lrelu(zl + zr)
        e = _dot(z, ae[...])
        s = jnp.max(e)
        ex = jnp.exp(e - s)
        den = _dot(pdt, ex)
        dend = _dot(pd, den)
        alpha = ex / (dend + 1e-16)
        aexp = _dot(alpha, hexp)
        out = _dot(pdt, zl * aexp) + bias[...]
        x = _elu(_layernorm(out, lg[...], lb[...]))
    gf = _dot(x, wf_ref[...]) + bf_ref[...]
    gb = _dot(x, wb_ref[...]) + bb_ref[...]
    for q in range(kq):
        gxf_ref[:, 0, q, :] = gf[q * tc:(q + 1) * tc, :]
        gxb_ref[:, 0, q, :] = gb[q * tc:(q + 1) * tc, :]


def _lstm_dirs(gxf_ref, gxb_ref, whf_ref, whb_ref, hf_s, cf_s, hb_s, cb_s,
               lh):
    i = pl.program_id(0)

    @pl.when(i == 0)
    def _init():
        z = jnp.zeros_like(hf_s)
        hf_s[...] = z
        cf_s[...] = z
        hb_s[...] = z
        cb_s[...] = z

    def one_dir(gx, h, c, whh):
        g = gx + _dot(h, whh)
        ig = jax.nn.sigmoid(g[:, 0 * lh:1 * lh])
        fg = jax.nn.sigmoid(g[:, 1 * lh:2 * lh])
        gg = jnp.tanh(g[:, 2 * lh:3 * lh])
        og = jax.nn.sigmoid(g[:, 3 * lh:4 * lh])
        c2 = fg * c + ig * gg
        h2 = og * jnp.tanh(c2)
        return h2, c2

    h2f, c2f = one_dir(gxf_ref[0], hf_s[...], cf_s[...], whf_ref[...])
    hf_s[...] = h2f
    cf_s[...] = c2f
    h2b, c2b = one_dir(gxb_ref[0], hb_s[...], cb_s[...], whb_ref[...])
    hb_s[...] = h2b
    cb_s[...] = c2b
    return h2f, h2b


def _lstm_step(gxf_ref, gxb_ref, whf_ref, whb_ref, hf_out, hb_out,
               hf_s, cf_s, hb_s, cb_s, *, lh):
    h2f, h2b = _lstm_dirs(gxf_ref, gxb_ref, whf_ref, whb_ref,
                          hf_s, cf_s, hb_s, cb_s, lh)
    hf_out[0] = h2f
    hb_out[0] = h2b


def _lstm_last_step(gxf_ref, gxb_ref, whf_ref, whb_ref, hf_out, hb_out,
                    hf_s, cf_s, hb_s, cb_s, *, lh):
    h2f, h2b = _lstm_dirs(gxf_ref, gxb_ref, whf_ref, whb_ref,
                          hf_s, cf_s, hb_s, cb_s, lh)
    hf_out[...] = h2f

    @pl.when(pl.program_id(0) == 0)
    def _store_bwd():
        hb_out[...] = h2b


def _mm_step(x_ref, w_ref, b_ref, o_ref):
    o_ref[0] = _dot(x_ref[0], w_ref[...]) + b_ref[...]


def _heads_step(t_ref, w1, b1, lg, lb, w2, b2, dw1, db1, dw2, db2,
                h_out, d_out):
    t = t_ref[...]
    h = jax.nn.relu(_dot(t, w1[...]) + b1[...])
    h = _layernorm(h, lg[...], lb[...])
    h_out[...] = jax.nn.sigmoid(_dot(h, w2[...]) + b2[...])
    d = jax.nn.relu(_dot(t, dw1[...]) + db1[...])
    d_out[...] = jnp.tanh(_dot(d, dw2[...]) + db2[...])


def _row(v):
    return v.reshape(1, -1)


@jax.jit
def kernel(x_sequence, edge_index, params):
    B, T, N, F = x_sequence.shape
    HEADS, HC = params['gat0_att'].shape
    H = HEADS * HC
    LH = params['lstm0_fwd_Whh'].shape[1]
    G = 4 * LH
    BN = B * N
    f32 = jnp.float32

    ar = jnp.arange(N, dtype=edge_index.dtype)
    src = jnp.concatenate([edge_index[0], ar])
    dst = jnp.concatenate([edge_index[1], ar])
    Ne = src.shape[0]
    cols = ar[None, :]
    Ps = (src[:, None] == cols).astype(f32)
    Pd = (dst[:, None] == cols).astype(f32)
    PdT = Pd.T

    eye = jnp.eye(HEADS, dtype=f32)
    hexp = jnp.repeat(eye, HC, axis=1)
    att_embs = [
        (params['gat%d_att' % i][:, :, None] * eye[:, None, :]).reshape(H, HEADS)
        for i in range(3)
    ]

    def full(shape):
        return pl.BlockSpec(shape, lambda t, b: tuple(0 for _ in shape))

    gat_in_specs = [
        pl.BlockSpec((1, 1, N, F), lambda t, b: (b, t, 0, 0)),
        full((Ne, N)), full((Ne, N)), full((N, Ne)),
    ]
    gat_args = [x_sequence, Ps, Pd, PdT]
    dins = [F, H, H]
    for i in range(3):
        gat_args += [params['gat%d_Wl' % i], params['gat%d_Wr' % i],
                     att_embs[i], _row(params['gat%d_b' % i]),
                     _row(params['ln%d_g' % i]), _row(params['ln%d_b' % i])]
        gat_in_specs += [full((dins[i], H)), full((dins[i], H)),
                         full((H, HEADS)), full((1, H)), full((1, H)),
                         full((1, H))]
    gat_args += [
        hexp,
        params['lstm0_fwd_Wih'].T,
        _row(params['lstm0_fwd_bih'] + params['lstm0_fwd_bhh']),
        params['lstm0_bwd_Wih'].T,
        _row(params['lstm0_bwd_bih'] + params['lstm0_bwd_bhh']),
    ]
    gat_in_specs += [full((HEADS, H)), full((H, G)), full((1, G)),
                     full((H, G)), full((1, G))]

    kq = N // T
    gxf, gxb = pl.pallas_call(
        functools.partial(_gat_step, tc=T, kq=kq),
        grid=(T, B),
        in_specs=gat_in_specs,
        out_specs=[pl.BlockSpec((T, 1, kq, G),
                                lambda t, b: (0, b * T + t, 0, 0))] * 2,
        out_shape=[jax.ShapeDtypeStruct((T, B * T, kq, G), f32)] * 2,
        compiler_params=pltpu.CompilerParams(
            dimension_semantics=("parallel", "parallel")),
    )(*gat_args)

    gxf = gxf.reshape(T, BN, G)
    gxb = gxb.reshape(T, BN, G)

    lstm0 = functools.partial(_lstm_step, lh=LH)
    hf, hb = pl.pallas_call(
        lstm0,
        grid=(T,),
        in_specs=[
            pl.BlockSpec((1, BN, G), lambda t: (t, 0, 0)),
            pl.BlockSpec((1, BN, G), lambda t: (T - 1 - t, 0, 0)),
            pl.BlockSpec((LH, G), lambda t: (0, 0)),
            pl.BlockSpec((LH, G), lambda t: (0, 0)),
        ],
        out_specs=[
            pl.BlockSpec((1, BN, LH), lambda t: (t, 0, 0)),
            pl.BlockSpec((1, BN, LH), lambda t: (T - 1 - t, 0, 0)),
        ],
        out_shape=[jax.ShapeDtypeStruct((T, BN, LH), f32)] * 2,
        scratch_shapes=[pltpu.VMEM((BN, LH), f32)] * 4,
    )(gxf, gxb, params['lstm0_fwd_Whh'].T, params['lstm0_bwd_Whh'].T)

    x1 = jnp.concatenate([hf, hb], axis=-1)
    w1cat = jnp.concatenate(
        [params['lstm1_fwd_Wih'].T, params['lstm1_bwd_Wih'].T], axis=-1)
    b1cat = jnp.concatenate([
        params['lstm1_fwd_bih'] + params['lstm1_fwd_bhh'],
        params['lstm1_bwd_bih'] + params['lstm1_bwd_bhh']]).reshape(1, 2 * G)
    gx1 = pl.pallas_call(
        _mm_step,
        grid=(T,),
        in_specs=[
            pl.BlockSpec((1, BN, 2 * LH), lambda t: (t, 0, 0)),
            pl.BlockSpec((2 * LH, 2 * G), lambda t: (0, 0)),
            pl.BlockSpec((1, 2 * G), lambda t: (0, 0)),
        ],
        out_specs=pl.BlockSpec((1, BN, 2 * G), lambda t: (t, 0, 0)),
        out_shape=jax.ShapeDtypeStruct((T, BN, 2 * G), f32),
        compiler_params=pltpu.CompilerParams(
            dimension_semantics=("parallel",)),
    )(x1, w1cat, b1cat)

    lstm1 = functools.partial(_lstm_last_step, lh=LH)
    hf1, hb1 = pl.pallas_call(
        lstm1,
        grid=(T,),
        in_specs=[
            pl.BlockSpec((1, BN, G), lambda t: (t, 0, 0)),
            pl.BlockSpec((1, BN, G), lambda t: (T - 1 - t, 0, 1)),
            pl.BlockSpec((LH, G), lambda t: (0, 0)),
            pl.BlockSpec((LH, G), lambda t: (0, 0)),
        ],
        out_specs=[
            pl.BlockSpec((BN, LH), lambda t: (0, 0)),
            pl.BlockSpec((BN, LH), lambda t: (0, 0)),
        ],
        out_shape=[jax.ShapeDtypeStruct((BN, LH), f32)] * 2,
        scratch_shapes=[pltpu.VMEM((BN, LH), f32)] * 4,
    )(gx1, gx1, params['lstm1_fwd_Whh'].T, params['lstm1_bwd_Whh'].T)

    temporal = jnp.concatenate([hf1, hb1], axis=-1)
    health, deg = pl.pallas_call(
        _heads_step,
        out_shape=[jax.ShapeDtypeStruct((BN, 1), f32)] * 2,
    )(temporal,
      params['hh_W1'], _row(params['hh_b1']),
      _row(params['hh_lng']), _row(params['hh_lnb']),
      params['hh_W2'], _row(params['hh_b2']),
      params['dh_W1'], _row(params['dh_b1']),
      params['dh_W2'], _row(params['dh_b2']))

    return health.reshape(B, N), deg.reshape(B, N)
